# baseline (speedup 1.0000x reference)
import functools

import jax
import jax.numpy as jnp
from jax.experimental import pallas as pl
import jax.experimental.pallas.tpu as pltpu

D = 768
F = 4 * D
RH = 2 * D
E = 8
K = 4
S = 2048
EPAD = 128

NEG = -3.0e38


def _router_kernel(x_ref, w1_ref, b1_ref, ws1_ref, bs1_ref, ws2_ref, bs2_ref,
                   ws3_ref, bs3_ref, w2_ref, b2_ref, mask_ref):
    x = x_ref[...]
    r1 = jnp.dot(x, w1_ref[...], preferred_element_type=jnp.float32) + b1_ref[...]
    a1 = jnp.dot(r1, ws1_ref[...], preferred_element_type=jnp.float32) + bs1_ref[...]
    a2 = jnp.dot(r1, ws2_ref[...], preferred_element_type=jnp.float32) + bs2_ref[...]
    h = jax.nn.silu(a1) * a2
    r2 = jnp.dot(h, ws3_ref[...], preferred_element_type=jnp.float32) + bs3_ref[...]
    logits = jnp.dot(r2, w2_ref[...],
                     preferred_element_type=jnp.float32) + b2_ref[...]
    lane = jax.lax.broadcasted_iota(jnp.int32, logits.shape, 1)
    p = logits
    mask = jnp.zeros_like(logits)
    for _ in range(K):
        m = jnp.max(p, axis=-1, keepdims=True)
        cand = p == m
        idx = jnp.where(cand, lane, EPAD)
        mi = jnp.min(idx, axis=-1, keepdims=True)
        sel = lane == mi
        mask = mask + jnp.where(sel, jnp.float32(1.0 / E), 0.0)
        p = jnp.where(sel, NEG, p)
    mask_ref[...] = mask


def _expert_kernel(x_ref, mask_ref, wf1_ref, bf1_ref, ws1_ref, bs1_ref,
                   ws2_ref, bs2_ref, ws3_ref, bs3_ref, wf2_ref, bf2_ref,
                   out_ref, h_ref, acc_ref, *, n_g):
    e = pl.program_id(0)
    ts = pl.program_id(1)
    g = pl.program_id(2)

    @pl.when((e == 0) & (ts == 0) & (g == 0))
    def _zero_out():
        out_ref[...] = jnp.zeros_like(out_ref)

    @pl.when(g == 0)
    def _compute_h():
        h = jnp.dot(x_ref[...], wf1_ref[0],
                    preferred_element_type=jnp.float32) + bf1_ref[0]
        h_ref[...] = h.astype(jnp.bfloat16)
        acc_ref[...] = jnp.zeros_like(acc_ref)

    h = h_ref[...]
    a1 = jnp.dot(h, ws1_ref[0], preferred_element_type=jnp.float32) + bs1_ref[0]
    a2 = jnp.dot(h, ws2_ref[0], preferred_element_type=jnp.float32) + bs2_ref[0]
    hid = (jax.nn.silu(a1) * a2).astype(jnp.bfloat16)
    acc_ref[...] += jnp.dot(hid, ws3_ref[0], preferred_element_type=jnp.float32)

    @pl.when(g == n_g - 1)
    def _finish():
        s = (acc_ref[...] + bs3_ref[0]).astype(jnp.bfloat16)
        o = jnp.dot(s, wf2_ref[0], preferred_element_type=jnp.float32) + bf2_ref[0]
        m = mask_ref[...]
        lane = jax.lax.broadcasted_iota(jnp.int32, m.shape, 1)
        w = jnp.sum(jnp.where(lane == e, m, 0.0), axis=1, keepdims=True)
        out_ref[pl.ds(ts * o.shape[0], o.shape[0]), :] += o * w


@jax.jit
def _moe(x, Wr1, br1, Wrs1, brs1, Wrs2, brs2, Wrs3, brs3, Wr2, br2,
         Wf1, bf1, Ws1, bs1, Ws2, bs2, Ws3, bs3, Wf2, bf2):
    xs = x.reshape(S, D)
    xb = xs.astype(jnp.bfloat16)
    bf16 = jnp.bfloat16

    w2p = jnp.zeros((RH, EPAD), jnp.float32).at[:, :E].set(Wr2.T)
    b2p = jnp.full((1, EPAD), NEG, jnp.float32).at[0, :E].set(br2)
    RT = 512
    mask = pl.pallas_call(
        _router_kernel,
        grid=(S // RT,),
        in_specs=[
            pl.BlockSpec((RT, D), lambda t: (t, 0)),
            pl.BlockSpec((D, RH), lambda t: (0, 0)),
            pl.BlockSpec((1, RH), lambda t: (0, 0)),
            pl.BlockSpec((RH, RH), lambda t: (0, 0)),
            pl.BlockSpec((1, RH), lambda t: (0, 0)),
            pl.BlockSpec((RH, RH), lambda t: (0, 0)),
            pl.BlockSpec((1, RH), lambda t: (0, 0)),
            pl.BlockSpec((RH, RH), lambda t: (0, 0)),
            pl.BlockSpec((1, RH), lambda t: (0, 0)),
            pl.BlockSpec((RH, EPAD), lambda t: (0, 0)),
            pl.BlockSpec((1, EPAD), lambda t: (0, 0)),
        ],
        out_specs=pl.BlockSpec((RT, EPAD), lambda t: (t, 0)),
        out_shape=jax.ShapeDtypeStruct((S, EPAD), jnp.float32),
    )(xs, Wr1.T, br1.reshape(1, RH),
      Wrs1.T, brs1.reshape(1, RH),
      Wrs2.T, brs2.reshape(1, RH),
      Wrs3.T, brs3.reshape(1, RH),
      w2p, b2p)

    ROWS = 512
    GB = 256
    n_ts = S // ROWS
    n_g = F // GB
    Wf1t = jnp.swapaxes(Wf1, 1, 2).astype(bf16)
    Ws1t = jnp.swapaxes(Ws1, 1, 2).astype(bf16)
    Ws2t = jnp.swapaxes(Ws2, 1, 2).astype(bf16)
    Ws3t = jnp.swapaxes(Ws3, 1, 2).astype(bf16)
    Wf2t = jnp.swapaxes(Wf2, 1, 2).astype(bf16)

    out = pl.pallas_call(
        functools.partial(_expert_kernel, n_g=n_g),
        grid=(E, n_ts, n_g),
        in_specs=[
            pl.BlockSpec((ROWS, D), lambda e, ts, g: (ts, 0)),
            pl.BlockSpec((ROWS, EPAD), lambda e, ts, g: (ts, 0)),
            pl.BlockSpec((1, D, F), lambda e, ts, g: (e, 0, 0)),
            pl.BlockSpec((1, 1, F), lambda e, ts, g: (e, 0, 0)),
            pl.BlockSpec((1, F, GB), lambda e, ts, g: (e, 0, g)),
            pl.BlockSpec((1, 1, GB), lambda e, ts, g: (e, 0, g)),
            pl.BlockSpec((1, F, GB), lambda e, ts, g: (e, 0, g)),
            pl.BlockSpec((1, 1, GB), lambda e, ts, g: (e, 0, g)),
            pl.BlockSpec((1, GB, F), lambda e, ts, g: (e, g, 0)),
            pl.BlockSpec((1, 1, F), lambda e, ts, g: (e, 0, 0)),
            pl.BlockSpec((1, F, D), lambda e, ts, g: (e, 0, 0)),
            pl.BlockSpec((1, 1, D), lambda e, ts, g: (e, 0, 0)),
        ],
        out_specs=pl.BlockSpec((S, D), lambda e, ts, g: (0, 0)),
        out_shape=jax.ShapeDtypeStruct((S, D), jnp.float32),
        scratch_shapes=[
            pltpu.VMEM((ROWS, F), jnp.bfloat16),
            pltpu.VMEM((ROWS, F), jnp.float32),
        ],
    )(xb, mask, Wf1t, bf1.reshape(E, 1, F),
      Ws1t, bs1.reshape(E, 1, F), Ws2t, bs2.reshape(E, 1, F),
      Ws3t, bs3.reshape(E, 1, F), Wf2t, bf2.reshape(E, 1, D))
    return out.reshape(1, S, D)


def kernel(x, Wr1, br1, Wrs1, brs1, Wrs2, brs2, Wrs3, brs3, Wr2, br2,
           Wf1, bf1, Ws1, bs1, Ws2, bs2, Ws3, bs3, Wf2, bf2):
    return _moe(x, Wr1, br1, Wrs1, brs1, Wrs2, brs2, Wrs3, brs3, Wr2, br2,
                Wf1, bf1, Ws1, bs1, Ws2, bs2, Ws3, bs3, Wf2, bf2)
